# baseline (speedup 1.0000x reference)
import functools

import jax
import jax.numpy as jnp
from jax import lax
from jax.experimental import pallas as pl
from jax.experimental.pallas import tpu as pltpu
from jax.experimental.pallas import tpu_sc as plsc

B, H = 128, 1024

NC, NS = 2, 16
NW_USED = 16
ROWS = B // NW_USED


@functools.cache
def _make_sc_gather():
    mesh = plsc.VectorSubcoreMesh(core_axis_name="c", subcore_axis_name="s")

    @functools.partial(
        pl.kernel,
        mesh=mesh,
        out_type=jax.ShapeDtypeStruct((B, H), jnp.float32),
        scratch_types=[
            pltpu.VMEM((ROWS,), jnp.int32),
            pltpu.VMEM((ROWS, H), jnp.float32),
            pltpu.SemaphoreType.DMA,
        ],
    )
    def _sc_gather(table_hbm, idx_hbm, out_hbm, idx_v, rows_v, sem):
        wid = lax.axis_index("s") * NC + lax.axis_index("c")

        @pl.when(wid < NW_USED)
        def _():
            base = wid * ROWS
            pltpu.sync_copy(idx_hbm.at[pl.ds(base, ROWS)], idx_v)
            pltpu.async_copy(table_hbm.at[idx_v], rows_v, sem).wait()
            pltpu.sync_copy(rows_v, out_hbm.at[pl.ds(base, ROWS)])

    return _sc_gather


G1 = 4
HB1 = 4 * H // G1

_dn = (((1,), (1,)), ((), ()))


def _hh_body(h0_ref, whh_ref, bih_ref, bhh_ref, acc_out):
    acc = lax.dot_general(h0_ref[...], whh_ref[...], _dn,
                          preferred_element_type=jnp.float32)
    acc_out[...] = acc + bih_ref[...] + bhh_ref[...]


_hh_part = pl.pallas_call(
    _hh_body,
    grid=(G1,),
    in_specs=[
        pl.BlockSpec((B, H), lambda k: (0, 0)),
        pl.BlockSpec((HB1, H), lambda k: (k, 0)),
        pl.BlockSpec((1, HB1), lambda k: (0, k)),
        pl.BlockSpec((1, HB1), lambda k: (0, k)),
    ],
    out_specs=pl.BlockSpec((B, HB1), lambda k: (0, k)),
    out_shape=jax.ShapeDtypeStruct((B, 4 * H), jnp.float32),
    compiler_params=pltpu.CompilerParams(dimension_semantics=("arbitrary",)),
)


G2 = 8
HB2 = 4 * H // G2


def _ih_body(emb_ref, c0_ref, wih_ref, acc_ref, h_out, c_out, act_ref):
    k = pl.program_id(0)
    pre = lax.dot_general(emb_ref[...], wih_ref[...], _dn,
                          preferred_element_type=jnp.float32)
    pre += acc_ref[...]
    quarter = k // (G2 // 4)
    act = jnp.where(quarter == 2, jnp.tanh(pre), jax.nn.sigmoid(pre))
    act_ref[:, pl.ds(k * HB2, HB2)] = act

    @pl.when(k == G2 - 1)
    def _():
        i = act_ref[:, 0:H]
        f = act_ref[:, H:2 * H]
        g = act_ref[:, 2 * H:3 * H]
        o = act_ref[:, 3 * H:4 * H]
        c = f * c0_ref[...] + i * g
        c_out[...] = c
        h_out[...] = o * jnp.tanh(c)


_ih_part = pl.pallas_call(
    _ih_body,
    grid=(G2,),
    in_specs=[
        pl.BlockSpec((B, H), lambda k: (0, 0)),
        pl.BlockSpec((B, H), lambda k: (0, 0)),
        pl.BlockSpec((HB2, H), lambda k: (k, 0)),
        pl.BlockSpec((B, HB2), lambda k: (0, k)),
    ],
    out_specs=[
        pl.BlockSpec((B, H), lambda k: (0, 0)),
        pl.BlockSpec((B, H), lambda k: (0, 0)),
    ],
    out_shape=[
        jax.ShapeDtypeStruct((B, H), jnp.float32),
        jax.ShapeDtypeStruct((B, H), jnp.float32),
    ],
    scratch_shapes=[pltpu.VMEM((B, 4 * H), jnp.float32)],
    compiler_params=pltpu.CompilerParams(dimension_semantics=("arbitrary",)),
)


def kernel(x, hidden, cell, table, W_ih, W_hh, b_ih, b_hh):
    emb = _make_sc_gather()(table, x)
    acc = _hh_part(hidden[0], W_hh,
                   b_ih.reshape(1, 4 * H), b_hh.reshape(1, 4 * H))
    h, c = _ih_part(emb, cell[0], W_ih, acc)
    return (h[None], h[None], c[None])


# speedup vs baseline: 1.7161x; 1.7161x over previous
import jax
import jax.numpy as jnp
from jax import lax
from jax.experimental import pallas as pl
from jax.experimental.pallas import tpu as pltpu

B, H = 128, 1024
G = 8
HB = 4 * H // G
RPS = B // G

_dn = (((1,), (1,)), ((), ()))


def _body(x_ref, h0_ref, c0_ref, whh_ref, wih_ref, bih_ref, bhh_ref,
          table_ref, h_out, c_out, acc_ref, emb_ref, sem):
    k = pl.program_id(0)

    @pl.when(k < G)
    def _phase_a():
        for r in range(RPS):
            b = k * RPS + r
            pltpu.make_async_copy(
                table_ref.at[pl.ds(x_ref[b], 1)],
                emb_ref.at[pl.ds(b, 1)],
                sem,
            ).start()
        acc = lax.dot_general(h0_ref[...], whh_ref[...], _dn,
                              preferred_element_type=jnp.float32)
        acc_ref[:, pl.ds(k * HB, HB)] = acc + bih_ref[...] + bhh_ref[...]

    @pl.when(k >= G)
    def _phase_b():
        j = k - G

        @pl.when(j == 0)
        def _wait_gather():
            for r in range(B):
                pltpu.make_async_copy(
                    table_ref.at[pl.ds(x_ref[r], 1)],
                    emb_ref.at[pl.ds(r, 1)],
                    sem,
                ).wait()

        pre = lax.dot_general(emb_ref[...], wih_ref[...], _dn,
                              preferred_element_type=jnp.float32)
        pre += acc_ref[:, pl.ds(j * HB, HB)]
        quarter = j // (G // 4)
        act = jnp.where(quarter == 2, jnp.tanh(pre), jax.nn.sigmoid(pre))
        acc_ref[:, pl.ds(j * HB, HB)] = act

        @pl.when(quarter == 3)
        def _finish():
            col = (j - 3 * (G // 4)) * HB
            i = acc_ref[:, pl.ds(col, HB)]
            f = acc_ref[:, pl.ds(H + col, HB)]
            g = acc_ref[:, pl.ds(2 * H + col, HB)]
            c = f * c0_ref[:, pl.ds(col, HB)] + i * g
            c_out[:, pl.ds(col, HB)] = c
            h_out[:, pl.ds(col, HB)] = act * jnp.tanh(c)


_lstm = pl.pallas_call(
    _body,
    grid=(2 * G,),
    in_specs=[
        pl.BlockSpec(memory_space=pltpu.SMEM),
        pl.BlockSpec((B, H), lambda k: (0, 0)),
        pl.BlockSpec((B, H), lambda k: (0, 0)),
        pl.BlockSpec((HB, H),
                     lambda k: (jnp.minimum(k, G - 1), 0)),
        pl.BlockSpec((HB, H),
                     lambda k: (jnp.maximum(k - G, 0), 0)),
        pl.BlockSpec((1, HB),
                     lambda k: (0, jnp.minimum(k, G - 1))),
        pl.BlockSpec((1, HB),
                     lambda k: (0, jnp.minimum(k, G - 1))),
        pl.BlockSpec(memory_space=pl.ANY),
    ],
    out_specs=[
        pl.BlockSpec((B, H), lambda k: (0, 0)),
        pl.BlockSpec((B, H), lambda k: (0, 0)),
    ],
    out_shape=[
        jax.ShapeDtypeStruct((B, H), jnp.float32),
        jax.ShapeDtypeStruct((B, H), jnp.float32),
    ],
    scratch_shapes=[
        pltpu.VMEM((B, 4 * H), jnp.float32),
        pltpu.VMEM((B, H), jnp.float32),
        pltpu.SemaphoreType.DMA,
    ],
    compiler_params=pltpu.CompilerParams(
        dimension_semantics=("arbitrary",)),
)


def kernel(x, hidden, cell, table, W_ih, W_hh, b_ih, b_hh):
    h, c = _lstm(x, hidden[0], cell[0], W_hh, W_ih,
                 b_ih.reshape(1, 4 * H), b_hh.reshape(1, 4 * H), table)
    return (h[None], h[None], c[None])


# speedup vs baseline: 1.7256x; 1.0055x over previous
import jax
import jax.numpy as jnp
from jax import lax
from jax.experimental import pallas as pl
from jax.experimental.pallas import tpu as pltpu

B, H = 128, 1024
G = 8
HB = 4 * H // G
RPS = B // G

_dn = (((1,), (1,)), ((), ()))


def _wih_copy(wih_hbm, wih_buf, sem2, jj):
    return pltpu.make_async_copy(
        wih_hbm.at[pl.ds(jj * HB, HB)],
        wih_buf.at[lax.rem(jj, 2)],
        sem2,
    )


def _body(x_ref, h0_ref, c0_ref, whh_ref, bih_ref, bhh_ref,
          table_ref, wih_hbm, h_out, c_out,
          acc_ref, emb_ref, wih_buf, sem, sem2):
    k = pl.program_id(0)

    @pl.when((k >= G - 1) & (k <= 2 * G - 2))
    def _issue_wih():
        _wih_copy(wih_hbm, wih_buf, sem2, k - (G - 1)).start()

    @pl.when(k < G)
    def _phase_a():
        for r in range(RPS):
            b = k * RPS + r
            pltpu.make_async_copy(
                table_ref.at[pl.ds(x_ref[b], 1)],
                emb_ref.at[pl.ds(b, 1)],
                sem,
            ).start()
        acc = lax.dot_general(h0_ref[...], whh_ref[...], _dn,
                              preferred_element_type=jnp.float32)
        acc_ref[:, pl.ds(k * HB, HB)] = acc + bih_ref[...] + bhh_ref[...]

    @pl.when(k >= G)
    def _phase_b():
        j = k - G

        @pl.when(j == 0)
        def _wait_gather():
            for r in range(B):
                pltpu.make_async_copy(
                    table_ref.at[pl.ds(x_ref[r], 1)],
                    emb_ref.at[pl.ds(r, 1)],
                    sem,
                ).wait()

        _wih_copy(wih_hbm, wih_buf, sem2, j).wait()
        wih = wih_buf[lax.rem(j, 2)]
        pre = lax.dot_general(emb_ref[...], wih, _dn,
                              preferred_element_type=jnp.float32)
        pre += acc_ref[:, pl.ds(j * HB, HB)]
        quarter = j // (G // 4)
        act = jnp.where(quarter == 2, jnp.tanh(pre), jax.nn.sigmoid(pre))
        acc_ref[:, pl.ds(j * HB, HB)] = act

        @pl.when(quarter == 3)
        def _finish():
            col = (j - 3 * (G // 4)) * HB
            i = acc_ref[:, pl.ds(col, HB)]
            f = acc_ref[:, pl.ds(H + col, HB)]
            g = acc_ref[:, pl.ds(2 * H + col, HB)]
            c = f * c0_ref[:, pl.ds(col, HB)] + i * g
            c_out[:, pl.ds(col, HB)] = c
            h_out[:, pl.ds(col, HB)] = act * jnp.tanh(c)


def _mk(interpret=False):
    return pl.pallas_call(
        _body,
        grid=(2 * G,),
        in_specs=[
            pl.BlockSpec(memory_space=pltpu.SMEM),
            pl.BlockSpec((B, H), lambda k: (0, 0)),
            pl.BlockSpec((B, H), lambda k: (0, 0)),
            pl.BlockSpec((HB, H),
                         lambda k: (jnp.minimum(k, G - 1), 0)),
            pl.BlockSpec((1, HB),
                         lambda k: (0, jnp.minimum(k, G - 1))),
            pl.BlockSpec((1, HB),
                         lambda k: (0, jnp.minimum(k, G - 1))),
            pl.BlockSpec(memory_space=pl.ANY),
            pl.BlockSpec(memory_space=pl.ANY),
        ],
        out_specs=[
            pl.BlockSpec((B, H), lambda k: (0, 0)),
            pl.BlockSpec((B, H), lambda k: (0, 0)),
        ],
        out_shape=[
            jax.ShapeDtypeStruct((B, H), jnp.float32),
            jax.ShapeDtypeStruct((B, H), jnp.float32),
        ],
        scratch_shapes=[
            pltpu.VMEM((B, 4 * H), jnp.float32),
            pltpu.VMEM((B, H), jnp.float32),
            pltpu.VMEM((2, HB, H), jnp.float32),
            pltpu.SemaphoreType.DMA,
            pltpu.SemaphoreType.DMA,
        ],
        compiler_params=pltpu.CompilerParams(
            dimension_semantics=("arbitrary",)),
        interpret=interpret,
    )


_lstm = _mk()


def kernel(x, hidden, cell, table, W_ih, W_hh, b_ih, b_hh):
    h, c = _lstm(x, hidden[0], cell[0], W_hh,
                 b_ih.reshape(1, 4 * H), b_hh.reshape(1, 4 * H),
                 table, W_ih)
    return (h[None], h[None], c[None])


# speedup vs baseline: 1.8996x; 1.1009x over previous
import jax
import jax.numpy as jnp
from jax import lax
from jax.experimental import pallas as pl
from jax.experimental.pallas import tpu as pltpu

B, H = 128, 1024
G = 8
HB = 4 * H // G
RPS = B // G

_dn = (((1,), (1,)), ((), ()))


def _wih_copy(wih_hbm, wih_buf, sem2, jj):
    return pltpu.make_async_copy(
        wih_hbm.at[pl.ds(jj * HB, HB)],
        wih_buf.at[lax.rem(jj, 4)],
        sem2,
    )


def _body(x_ref, h0_ref, c0_ref, whh_ref, bih_ref, bhh_ref,
          table_ref, wih_hbm, h_out, c_out,
          acc_ref, emb_ref, wih_buf, sem, sem2):
    k = pl.program_id(0)

    @pl.when((k >= G - 3) & (k <= 2 * G - 4))
    def _issue_wih():
        _wih_copy(wih_hbm, wih_buf, sem2, k - (G - 3)).start()

    @pl.when(k < G)
    def _phase_a():
        for r in range(RPS):
            b = k * RPS + r
            pltpu.make_async_copy(
                table_ref.at[pl.ds(x_ref[b], 1)],
                emb_ref.at[pl.ds(b, 1)],
                sem,
            ).start()
        acc = lax.dot_general(h0_ref[...], whh_ref[...], _dn,
                              preferred_element_type=jnp.float32)
        acc_ref[:, pl.ds(k * HB, HB)] = acc + bih_ref[...] + bhh_ref[...]

    @pl.when(k >= G)
    def _phase_b():
        j = k - G

        @pl.when(j == 0)
        def _wait_gather():
            for r in range(B):
                pltpu.make_async_copy(
                    table_ref.at[pl.ds(x_ref[r], 1)],
                    emb_ref.at[pl.ds(r, 1)],
                    sem,
                ).wait()

        _wih_copy(wih_hbm, wih_buf, sem2, j).wait()
        wih = wih_buf[lax.rem(j, 4)]
        pre = lax.dot_general(emb_ref[...], wih, _dn,
                              preferred_element_type=jnp.float32)
        pre += acc_ref[:, pl.ds(j * HB, HB)]
        quarter = j // (G // 4)
        act = jnp.where(quarter == 2, jnp.tanh(pre), jax.nn.sigmoid(pre))
        acc_ref[:, pl.ds(j * HB, HB)] = act

        @pl.when(quarter == 3)
        def _finish():
            col = (j - 3 * (G // 4)) * HB
            i = acc_ref[:, pl.ds(col, HB)]
            f = acc_ref[:, pl.ds(H + col, HB)]
            g = acc_ref[:, pl.ds(2 * H + col, HB)]
            c = f * c0_ref[:, pl.ds(col, HB)] + i * g
            c_out[:, pl.ds(col, HB)] = c
            h_out[:, pl.ds(col, HB)] = act * jnp.tanh(c)


def _mk(interpret=False):
    return pl.pallas_call(
        _body,
        grid=(2 * G,),
        in_specs=[
            pl.BlockSpec(memory_space=pltpu.SMEM),
            pl.BlockSpec((B, H), lambda k: (0, 0)),
            pl.BlockSpec((B, H), lambda k: (0, 0)),
            pl.BlockSpec((HB, H),
                         lambda k: (jnp.minimum(k, G - 1), 0)),
            pl.BlockSpec((1, HB),
                         lambda k: (0, jnp.minimum(k, G - 1))),
            pl.BlockSpec((1, HB),
                         lambda k: (0, jnp.minimum(k, G - 1))),
            pl.BlockSpec(memory_space=pl.ANY),
            pl.BlockSpec(memory_space=pl.ANY),
        ],
        out_specs=[
            pl.BlockSpec((B, H), lambda k: (0, 0)),
            pl.BlockSpec((B, H), lambda k: (0, 0)),
        ],
        out_shape=[
            jax.ShapeDtypeStruct((B, H), jnp.float32),
            jax.ShapeDtypeStruct((B, H), jnp.float32),
        ],
        scratch_shapes=[
            pltpu.VMEM((B, 4 * H), jnp.float32),
            pltpu.VMEM((B, H), jnp.float32),
            pltpu.VMEM((4, HB, H), jnp.float32),
            pltpu.SemaphoreType.DMA,
            pltpu.SemaphoreType.DMA,
        ],
        compiler_params=pltpu.CompilerParams(
            dimension_semantics=("arbitrary",)),
        interpret=interpret,
    )


_lstm = _mk()


def kernel(x, hidden, cell, table, W_ih, W_hh, b_ih, b_hh):
    h, c = _lstm(x, hidden[0], cell[0], W_hh,
                 b_ih.reshape(1, 4 * H), b_hh.reshape(1, 4 * H),
                 table, W_ih)
    return (h[None], h[None], c[None])


# speedup vs baseline: 2.1282x; 1.1203x over previous
import jax
import jax.numpy as jnp
from jax import lax
from jax.experimental import pallas as pl
from jax.experimental.pallas import tpu as pltpu

B, H = 128, 1024
G = 8
HB = 4 * H // G
RPS = B // G
ND = 4

_dn = (((1,), (1,)), ((), ()))


def _w_copy(w_hbm, w_buf, sem, blk):
    return pltpu.make_async_copy(
        w_hbm.at[pl.ds(blk * HB, HB)],
        w_buf.at[lax.rem(blk, ND)],
        sem,
    )


def _body(x_ref, h0_ref, c0_ref, bih_ref, bhh_ref,
          table_ref, whh_hbm, wih_hbm, h_out, c_out,
          acc_ref, emb_ref, whh_buf, wih_buf, sem, semh, semi):
    k = pl.program_id(0)

    @pl.when(k == 0)
    def _prime_whh():
        for blk in range(ND):
            _w_copy(whh_hbm, whh_buf, semh, blk).start()

    @pl.when((k >= 1) & (k <= G - ND))
    def _issue_whh():
        _w_copy(whh_hbm, whh_buf, semh, k + ND - 1).start()

    @pl.when((k >= G - 3) & (k <= 2 * G - 4))
    def _issue_wih():
        _w_copy(wih_hbm, wih_buf, semi, k - (G - 3)).start()

    @pl.when(k < G)
    def _phase_a():
        for r in range(RPS):
            b = k * RPS + r
            pltpu.make_async_copy(
                table_ref.at[pl.ds(x_ref[b], 1)],
                emb_ref.at[pl.ds(b, 1)],
                sem,
            ).start()
        _w_copy(whh_hbm, whh_buf, semh, k).wait()
        whh = whh_buf[lax.rem(k, ND)]
        acc = lax.dot_general(h0_ref[...], whh, _dn,
                              preferred_element_type=jnp.float32)
        acc_ref[:, pl.ds(k * HB, HB)] = acc + bih_ref[...] + bhh_ref[...]

    @pl.when(k >= G)
    def _phase_b():
        j = k - G

        @pl.when(j == 0)
        def _wait_gather():
            for r in range(B):
                pltpu.make_async_copy(
                    table_ref.at[pl.ds(x_ref[r], 1)],
                    emb_ref.at[pl.ds(r, 1)],
                    sem,
                ).wait()

        _w_copy(wih_hbm, wih_buf, semi, j).wait()
        wih = wih_buf[lax.rem(j, ND)]
        pre = lax.dot_general(emb_ref[...], wih, _dn,
                              preferred_element_type=jnp.float32)
        pre += acc_ref[:, pl.ds(j * HB, HB)]
        quarter = j // (G // 4)
        act = jnp.where(quarter == 2, jnp.tanh(pre), jax.nn.sigmoid(pre))
        acc_ref[:, pl.ds(j * HB, HB)] = act

        @pl.when(quarter == 3)
        def _finish():
            col = (j - 3 * (G // 4)) * HB
            i = acc_ref[:, pl.ds(col, HB)]
            f = acc_ref[:, pl.ds(H + col, HB)]
            g = acc_ref[:, pl.ds(2 * H + col, HB)]
            c = f * c0_ref[:, pl.ds(col, HB)] + i * g
            c_out[:, pl.ds(col, HB)] = c
            h_out[:, pl.ds(col, HB)] = act * jnp.tanh(c)


def _mk(interpret=False):
    return pl.pallas_call(
        _body,
        grid=(2 * G,),
        in_specs=[
            pl.BlockSpec(memory_space=pltpu.SMEM),
            pl.BlockSpec((B, H), lambda k: (0, 0)),
            pl.BlockSpec((B, H), lambda k: (0, 0)),
            pl.BlockSpec((1, HB),
                         lambda k: (0, jnp.minimum(k, G - 1))),
            pl.BlockSpec((1, HB),
                         lambda k: (0, jnp.minimum(k, G - 1))),
            pl.BlockSpec(memory_space=pl.ANY),
            pl.BlockSpec(memory_space=pl.ANY),
            pl.BlockSpec(memory_space=pl.ANY),
        ],
        out_specs=[
            pl.BlockSpec((B, H), lambda k: (0, 0)),
            pl.BlockSpec((B, H), lambda k: (0, 0)),
        ],
        out_shape=[
            jax.ShapeDtypeStruct((B, H), jnp.float32),
            jax.ShapeDtypeStruct((B, H), jnp.float32),
        ],
        scratch_shapes=[
            pltpu.VMEM((B, 4 * H), jnp.float32),
            pltpu.VMEM((B, H), jnp.float32),
            pltpu.VMEM((ND, HB, H), jnp.float32),
            pltpu.VMEM((ND, HB, H), jnp.float32),
            pltpu.SemaphoreType.DMA,
            pltpu.SemaphoreType.DMA,
            pltpu.SemaphoreType.DMA,
        ],
        compiler_params=pltpu.CompilerParams(
            dimension_semantics=("arbitrary",)),
        interpret=interpret,
    )


_lstm = _mk()


def kernel(x, hidden, cell, table, W_ih, W_hh, b_ih, b_hh):
    h, c = _lstm(x, hidden[0], cell[0],
                 b_ih.reshape(1, 4 * H), b_hh.reshape(1, 4 * H),
                 table, W_hh, W_ih)
    return (h[None], h[None], c[None])
